# Initial kernel scaffold; baseline (speedup 1.0000x reference)
import functools

import jax
import jax.numpy as jnp
from jax import lax
from jax.experimental import pallas as pl
from jax.experimental.pallas import tpu as pltpu
from jax.experimental.pallas import tpu_sc as plsc

N_NODES_C = 10000
N_EDGES_C = 320000
D_FEAT_C = 128
D_EDGE_C = 16
HID = 8

NC = 2
NS = 16
L = 16

N_PAIRS = NC * NS // 2
E_PER_PAIR = N_EDGES_C // N_PAIRS
SUB = 2000
NSUB = E_PER_PAIR // SUB


def _mm_ab_body(x_ref, w_ref, o_ref):
    o_ref[...] = jnp.dot(x_ref[...], w_ref[...], preferred_element_type=jnp.float32)


def _mm_c_body(ea_ref, w_ref, b_ref, o_ref):
    o_ref[...] = (
        jnp.dot(ea_ref[...], w_ref[...], preferred_element_type=jnp.float32)
        + b_ref[...]
    )


_C_BLK = 16000


def _sc_edge_mlp(tab_a, tab_b, row_hbm, col_hbm, c_hbm, w2b_hbm, b2b_hbm,
                 out_hbm,
                 tabA_v, tabB_v, rowv, colv, cv, sv, tmpv, w2_v, b2_v, shared):
    c_id = lax.axis_index("c")
    s_id = lax.axis_index("s")
    pair = s_id // 2
    jh = s_id % 2
    ec = c_id * (NS // 2) + pair
    base = ec * E_PER_PAIR

    pltpu.sync_copy(tab_a.at[jh], tabA_v)
    pltpu.sync_copy(tab_b.at[jh], tabB_v)
    pltpu.sync_copy(w2b_hbm.at[pl.ds(jh * 4, 4)], w2_v)
    pltpu.sync_copy(b2b_hbm, b2_v)

    w2r = [w2_v[j] for j in range(4)]
    b2r = b2_v[...]
    zero16 = jnp.zeros((L,), jnp.float32)
    lane8 = lax.iota(jnp.int32, L) * 8

    def sub_chunk(k, carry):
        eb = base + k * SUB
        pltpu.sync_copy(row_hbm.at[pl.ds(eb, SUB)], rowv)
        pltpu.sync_copy(col_hbm.at[pl.ds(eb, SUB)], colv)
        pltpu.sync_copy(c_hbm.at[pl.ds(eb * 8, SUB * 8)], cv)

        def inner(i, carry2):
            r16 = rowv[pl.ds(i * L, L)] * 4
            q16 = colv[pl.ds(i * L, L)] * 4
            cb = lane8 + i * (8 * L)
            s_acc = zero16
            for jl in range(4):
                ag = plsc.load_gather(tabA_v, [r16 + jl])
                bg = plsc.load_gather(tabB_v, [q16 + jl])
                cg = plsc.load_gather(cv, [cb + jl])
                pre = jnp.maximum(ag + bg + cg, 0.0)
                s_acc = s_acc + pre * w2r[jl]
            sv[pl.ds(i * L, L)] = s_acc
            return carry2

        lax.fori_loop(0, SUB // L, inner, 0)

        slot = lax.rem(k, 2)

        @pl.when(jh == 1)
        def _():
            pltpu.sync_copy(sv, shared.at[slot, pair])

        plsc.subcore_barrier()

        @pl.when(jh == 0)
        def _():
            pltpu.sync_copy(shared.at[slot, pair], tmpv)

            def fin(i, carry2):
                v = sv[pl.ds(i * L, L)] + tmpv[pl.ds(i * L, L)] + b2r
                o = 1.0 / (1.0 + jnp.exp(-v))
                tmpv[pl.ds(i * L, L)] = o
                return carry2

            lax.fori_loop(0, SUB // L, fin, 0)
            pltpu.sync_copy(tmpv, out_hbm.at[pl.ds(eb, SUB)])

        return carry

    lax.fori_loop(0, NSUB, sub_chunk, 0)


_sc_call = functools.partial(
    pl.kernel,
    out_type=jax.ShapeDtypeStruct((N_EDGES_C,), jnp.float32),
    mesh=plsc.VectorSubcoreMesh(core_axis_name="c", subcore_axis_name="s"),
    scratch_types=[
        pltpu.VMEM((40000,), jnp.float32),
        pltpu.VMEM((40000,), jnp.float32),
        pltpu.VMEM((SUB,), jnp.int32),
        pltpu.VMEM((SUB,), jnp.int32),
        pltpu.VMEM((SUB * 8,), jnp.float32),
        pltpu.VMEM((SUB,), jnp.float32),
        pltpu.VMEM((SUB,), jnp.float32),
        pltpu.VMEM((4, L), jnp.float32),
        pltpu.VMEM((L,), jnp.float32),
        pltpu.VMEM_SHARED((2, NS // 2, SUB), jnp.float32),
    ],
)


def kernel(xn, edge_index, edge_attr, W1, b1, W2, b2):
    W1a = W1[:D_FEAT_C, :]
    W1b = W1[D_FEAT_C:2 * D_FEAT_C, :]
    W1c = W1[2 * D_FEAT_C:, :]
    Wab = jnp.concatenate([W1a, W1b], axis=1)

    AB = pl.pallas_call(
        _mm_ab_body,
        out_shape=jax.ShapeDtypeStruct((N_NODES_C, 2 * HID), jnp.float32),
    )(xn, Wab)

    C = pl.pallas_call(
        _mm_c_body,
        grid=(N_EDGES_C // _C_BLK,),
        in_specs=[
            pl.BlockSpec((_C_BLK, D_EDGE_C), lambda i: (i, 0)),
            pl.BlockSpec((D_EDGE_C, HID), lambda i: (0, 0)),
            pl.BlockSpec((1, HID), lambda i: (0, 0)),
        ],
        out_specs=pl.BlockSpec((_C_BLK, HID), lambda i: (i, 0)),
        out_shape=jax.ShapeDtypeStruct((N_EDGES_C, HID), jnp.float32),
    )(edge_attr, W1c, b1.reshape(1, HID))

    tab_a = jnp.stack([AB[:, 0:4].reshape(-1), AB[:, 4:8].reshape(-1)])
    tab_b = jnp.stack([AB[:, 8:12].reshape(-1), AB[:, 12:16].reshape(-1)])

    row = edge_index[0].astype(jnp.int32)
    col = edge_index[1].astype(jnp.int32)
    c_flat = C.reshape(-1)
    w2b = jnp.broadcast_to(W2.reshape(HID, 1), (HID, L))
    b2b = jnp.broadcast_to(b2, (L,))

    out = _sc_call(_sc_edge_mlp)(tab_a, tab_b, row, col, c_flat, w2b, b2b)
    return out.reshape(N_EDGES_C, 1)


# speedup vs baseline: 3.2009x; 3.2009x over previous
import functools

import jax
import jax.numpy as jnp
from jax import lax
from jax.experimental import pallas as pl
from jax.experimental.pallas import tpu as pltpu
from jax.experimental.pallas import tpu_sc as plsc

N_NODES_C = 10000
N_EDGES_C = 320000
D_FEAT_C = 128
D_EDGE_C = 16
HID = 8

NC = 2
NS = 16
L = 16

N_PAIRS = NC * NS // 2
E_PER_PAIR = N_EDGES_C // N_PAIRS
SUB = 2000
NSUB = E_PER_PAIR // SUB


def _mm_ab_body(x_ref, w_ref, o_ref):
    o_ref[...] = jnp.dot(x_ref[...], w_ref[...],
                         preferred_element_type=jnp.float32,
                         precision=lax.Precision.HIGHEST)


def _mm_c_body(ea_ref, w_ref, b_ref, o_ref):
    o_ref[...] = (
        jnp.dot(ea_ref[...], w_ref[...],
                preferred_element_type=jnp.float32,
                precision=lax.Precision.HIGHEST)
        + b_ref[...]
    )


_C_BLK = 2000


def _sc_edge_mlp(tab_a, tab_b, row_hbm, col_hbm, c_hbm, w2b_hbm,
                 out_hbm,
                 tabA_v, tabB_v, rowv, colv, cv, sv, w2_v):
    c_id = lax.axis_index("c")
    s_id = lax.axis_index("s")
    pair = s_id // 2
    jh = s_id % 2
    ec = c_id * (NS // 2) + pair
    base = ec * E_PER_PAIR

    pltpu.sync_copy(tab_a.at[jh], tabA_v)
    pltpu.sync_copy(tab_b.at[jh], tabB_v)
    pltpu.sync_copy(w2b_hbm.at[pl.ds(jh * 4, 4)], w2_v)

    w2r = [w2_v[j] for j in range(4)]
    zero16 = jnp.zeros((L,), jnp.float32)
    lane8 = lax.iota(jnp.int32, L) * 8

    def sub_chunk(k, carry):
        eb = base + k * SUB
        pltpu.sync_copy(row_hbm.at[pl.ds(eb, SUB)], rowv)
        pltpu.sync_copy(col_hbm.at[pl.ds(eb, SUB)], colv)
        pltpu.sync_copy(c_hbm.at[pl.ds(eb * 8, SUB * 8)], cv)

        def inner(i, carry2):
            r16 = rowv[pl.ds(i * L, L)] * 4
            q16 = colv[pl.ds(i * L, L)] * 4
            cb = lane8 + i * (8 * L) + jh * 4
            s_acc = zero16
            for jl in range(4):
                ag = plsc.load_gather(tabA_v, [r16 + jl])
                bg = plsc.load_gather(tabB_v, [q16 + jl])
                cg = plsc.load_gather(cv, [cb + jl])
                pre = jnp.maximum(ag + bg + cg, 0.0)
                s_acc = s_acc + pre * w2r[jl]
            sv[pl.ds(i * L, L)] = s_acc
            return carry2

        lax.fori_loop(0, SUB // L, inner, 0)

        pltpu.sync_copy(sv, out_hbm.at[pl.ds(jh * N_EDGES_C + eb, SUB)])

        return carry

    lax.fori_loop(0, NSUB, sub_chunk, 0)


_sc_call = functools.partial(
    pl.kernel,
    out_type=jax.ShapeDtypeStruct((2 * N_EDGES_C,), jnp.float32),
    mesh=plsc.VectorSubcoreMesh(core_axis_name="c", subcore_axis_name="s"),
    compiler_params=pltpu.CompilerParams(needs_layout_passes=False),
    scratch_types=[
        pltpu.VMEM((40000,), jnp.float32),
        pltpu.VMEM((40000,), jnp.float32),
        pltpu.VMEM((SUB,), jnp.int32),
        pltpu.VMEM((SUB,), jnp.int32),
        pltpu.VMEM((SUB * 8,), jnp.float32),
        pltpu.VMEM((SUB,), jnp.float32),
        pltpu.VMEM((4, L), jnp.float32),
    ],
)


def _combine_body(s0_ref, s1_ref, b2_ref, o_ref):
    v = s0_ref[...] + s1_ref[...] + b2_ref[...]
    o_ref[...] = jax.nn.sigmoid(v)


def kernel(xn, edge_index, edge_attr, W1, b1, W2, b2):
    W1a = W1[:D_FEAT_C, :]
    W1b = W1[D_FEAT_C:2 * D_FEAT_C, :]
    W1c = W1[2 * D_FEAT_C:, :]
    Wab = jnp.concatenate([W1a, W1b], axis=1)

    AB = pl.pallas_call(
        _mm_ab_body,
        out_shape=jax.ShapeDtypeStruct((N_NODES_C, 2 * HID), jnp.float32),
    )(xn, Wab)

    C = pl.pallas_call(
        _mm_c_body,
        grid=(N_EDGES_C // _C_BLK,),
        in_specs=[
            pl.BlockSpec((_C_BLK, D_EDGE_C), lambda i: (i, 0)),
            pl.BlockSpec((D_EDGE_C, HID), lambda i: (0, 0)),
            pl.BlockSpec((1, HID), lambda i: (0, 0)),
        ],
        out_specs=pl.BlockSpec((_C_BLK, HID), lambda i: (i, 0)),
        out_shape=jax.ShapeDtypeStruct((N_EDGES_C, HID), jnp.float32),
    )(edge_attr, W1c, b1.reshape(1, HID))

    tab_a = jnp.stack([AB[:, 0:4].reshape(-1), AB[:, 4:8].reshape(-1)])
    tab_b = jnp.stack([AB[:, 8:12].reshape(-1), AB[:, 12:16].reshape(-1)])

    row = edge_index[0].astype(jnp.int32)
    col = edge_index[1].astype(jnp.int32)
    c_flat = C.reshape(-1)
    w2b = jnp.broadcast_to(W2.reshape(HID, 1), (HID, L))

    s01 = _sc_call(_sc_edge_mlp)(tab_a, tab_b, row, col, c_flat, w2b)
    s0 = s01[:N_EDGES_C].reshape(2500, 128)
    s1 = s01[N_EDGES_C:].reshape(2500, 128)

    out = pl.pallas_call(
        _combine_body,
        out_shape=jax.ShapeDtypeStruct((2500, 128), jnp.float32),
    )(s0, s1, b2.reshape(1, 1))
    return out.reshape(N_EDGES_C, 1)


# speedup vs baseline: 5.5923x; 1.7471x over previous
import functools

import jax
import jax.numpy as jnp
from jax import lax
from jax.experimental import pallas as pl
from jax.experimental.pallas import tpu as pltpu
from jax.experimental.pallas import tpu_sc as plsc

N_NODES_C = 10000
N_EDGES_C = 320000
D_FEAT_C = 128
D_EDGE_C = 16
HID = 8

NC = 2
NS = 16
L = 16

N_PAIRS = NC * NS // 2
E_PER_PAIR = N_EDGES_C // N_PAIRS
SUB = 2000
NSUB = E_PER_PAIR // SUB


def _mm_ab_body(x_ref, w_ref, o_ref):
    o_ref[...] = jnp.dot(x_ref[...], w_ref[...],
                         preferred_element_type=jnp.float32,
                         precision=lax.Precision.HIGHEST)


def _mm_c_body(ea_ref, w_ref, b_ref, o_ref):
    o_ref[...] = (
        jnp.dot(ea_ref[...], w_ref[...],
                preferred_element_type=jnp.float32,
                precision=lax.Precision.HIGHEST)
        + b_ref[...]
    )


_C_BLK = 4000


def _sc_edge_mlp(tab_a, tab_b, row_hbm, col_hbm, c_hbm, w2b_hbm,
                 out_hbm,
                 tabA_v, tabB_v, rowv, colv, cv, sv, w2_v):
    c_id = lax.axis_index("c")
    s_id = lax.axis_index("s")
    pair = s_id // 2
    jh = s_id % 2
    ec = c_id * (NS // 2) + pair
    base = ec * E_PER_PAIR

    pltpu.sync_copy(tab_a.at[jh], tabA_v)
    pltpu.sync_copy(tab_b.at[jh], tabB_v)
    pltpu.sync_copy(w2b_hbm.at[pl.ds(jh * 4, 4)], w2_v)

    w2r = [w2_v[j] for j in range(4)]
    zero16 = jnp.zeros((L,), jnp.float32)
    lane8 = lax.iota(jnp.int32, L) * 8

    def sub_chunk(k, carry):
        eb = base + k * SUB
        pltpu.sync_copy(row_hbm.at[pl.ds(eb, SUB)], rowv)
        pltpu.sync_copy(col_hbm.at[pl.ds(eb, SUB)], colv)
        pltpu.sync_copy(c_hbm.at[pl.ds(eb * 8, SUB * 8)], cv)

        def inner(i, carry2):
            r16 = rowv[pl.ds(i * L, L)] * 4
            q16 = colv[pl.ds(i * L, L)] * 4
            cb = lane8 + i * (8 * L) + jh * 4
            s_acc = zero16
            for jl in range(4):
                ag = plsc.load_gather(tabA_v, [r16 + jl])
                bg = plsc.load_gather(tabB_v, [q16 + jl])
                cg = plsc.load_gather(cv, [cb + jl])
                pre = jnp.maximum(ag + bg + cg, 0.0)
                s_acc = s_acc + pre * w2r[jl]
            sv[pl.ds(i * L, L)] = s_acc
            return carry2

        lax.fori_loop(0, SUB // L, inner, 0)

        pltpu.sync_copy(sv, out_hbm.at[pl.ds(jh * N_EDGES_C + eb, SUB)])

        return carry

    lax.fori_loop(0, NSUB, sub_chunk, 0)


_sc_call = functools.partial(
    pl.kernel,
    out_type=jax.ShapeDtypeStruct((2 * N_EDGES_C,), jnp.float32),
    mesh=plsc.VectorSubcoreMesh(core_axis_name="c", subcore_axis_name="s"),
    compiler_params=pltpu.CompilerParams(needs_layout_passes=False),
    scratch_types=[
        pltpu.VMEM((40000,), jnp.float32),
        pltpu.VMEM((40000,), jnp.float32),
        pltpu.VMEM((SUB,), jnp.int32),
        pltpu.VMEM((SUB,), jnp.int32),
        pltpu.VMEM((SUB * 8,), jnp.float32),
        pltpu.VMEM((SUB,), jnp.float32),
        pltpu.VMEM((4, L), jnp.float32),
    ],
)


def _combine_body(s0_ref, s1_ref, b2_ref, o_ref):
    v = s0_ref[...] + s1_ref[...] + b2_ref[...]
    o_ref[...] = jax.nn.sigmoid(v)


def kernel(xn, edge_index, edge_attr, W1, b1, W2, b2):
    W1a = W1[:D_FEAT_C, :]
    W1b = W1[D_FEAT_C:2 * D_FEAT_C, :]
    W1c = W1[2 * D_FEAT_C:, :]
    Wab = jnp.concatenate([W1a, W1b], axis=1)

    AB = pl.pallas_call(
        _mm_ab_body,
        out_shape=jax.ShapeDtypeStruct((N_NODES_C, 2 * HID), jnp.float32),
    )(xn, Wab)

    ea_pack = edge_attr.reshape(N_EDGES_C // 8, 8 * D_EDGE_C)
    eye8 = jnp.eye(8, dtype=jnp.float32)
    w_bd = jnp.einsum("pq,kh->pkqh", eye8, W1c).reshape(8 * D_EDGE_C, 8 * HID)
    b_tile = jnp.tile(b1, 8).reshape(1, 8 * HID)
    C = pl.pallas_call(
        _mm_c_body,
        grid=(N_EDGES_C // 8 // _C_BLK,),
        in_specs=[
            pl.BlockSpec((_C_BLK, 8 * D_EDGE_C), lambda i: (i, 0)),
            pl.BlockSpec((8 * D_EDGE_C, 8 * HID), lambda i: (0, 0)),
            pl.BlockSpec((1, 8 * HID), lambda i: (0, 0)),
        ],
        out_specs=pl.BlockSpec((_C_BLK, 8 * HID), lambda i: (i, 0)),
        out_shape=jax.ShapeDtypeStruct((N_EDGES_C // 8, 8 * HID), jnp.float32),
    )(ea_pack, w_bd, b_tile)

    tab_a = jnp.stack([AB[:, 0:4].reshape(-1), AB[:, 4:8].reshape(-1)])
    tab_b = jnp.stack([AB[:, 8:12].reshape(-1), AB[:, 12:16].reshape(-1)])

    row = edge_index[0].astype(jnp.int32)
    col = edge_index[1].astype(jnp.int32)
    c_flat = C.reshape(-1)
    w2b = jnp.broadcast_to(W2.reshape(HID, 1), (HID, L))

    s01 = _sc_call(_sc_edge_mlp)(tab_a, tab_b, row, col, c_flat, w2b)
    s0 = s01[:N_EDGES_C].reshape(2500, 128)
    s1 = s01[N_EDGES_C:].reshape(2500, 128)

    out = pl.pallas_call(
        _combine_body,
        out_shape=jax.ShapeDtypeStruct((2500, 128), jnp.float32),
    )(s0, s1, b2.reshape(1, 1))
    return out.reshape(N_EDGES_C, 1)


# speedup vs baseline: 6.9976x; 1.2513x over previous
import functools

import jax
import jax.numpy as jnp
from jax import lax
from jax.experimental import pallas as pl
from jax.experimental.pallas import tpu as pltpu
from jax.experimental.pallas import tpu_sc as plsc

N_NODES_C = 10000
N_EDGES_C = 320000
D_FEAT_C = 128
D_EDGE_C = 16
HID = 8

NC = 2
NS = 16
L = 16

N_PAIRS = NC * NS // 2
E_PER_PAIR = N_EDGES_C // N_PAIRS
SUB = 2000
NSUB = E_PER_PAIR // SUB


def _mm_abt_body(x_ref, w_ref, o_ref):
    o_ref[...] = lax.dot_general(
        w_ref[...], x_ref[...],
        dimension_numbers=(((0,), (1,)), ((), ())),
        preferred_element_type=jnp.float32,
        precision=lax.Precision.HIGHEST)


def _mm_c_body(ea_ref, w_ref, b_ref, o_ref):
    o_ref[...] = (
        jnp.dot(ea_ref[...], w_ref[...],
                preferred_element_type=jnp.float32,
                precision=lax.Precision.HIGHEST)
        + b_ref[...]
    )


_C_BLK = 4000


def _sc_edge_mlp(abt_hbm, row_hbm, col_hbm, c_hbm, w2b_hbm,
                 out_hbm,
                 pa0, pa1, pa2, pa3, pb0, pb1, pb2, pb3,
                 rowv, colv, cv, sv, w2_v):
    c_id = lax.axis_index("c")
    s_id = lax.axis_index("s")
    pair = s_id // 2
    jh = s_id % 2
    ec = c_id * (NS // 2) + pair
    base = ec * E_PER_PAIR

    pa = [pa0, pa1, pa2, pa3]
    pb = [pb0, pb1, pb2, pb3]
    for jl in range(4):
        pltpu.sync_copy(abt_hbm.at[jh * 4 + jl], pa[jl])
        pltpu.sync_copy(abt_hbm.at[HID + jh * 4 + jl], pb[jl])
    pltpu.sync_copy(w2b_hbm.at[pl.ds(jh * 4, 4)], w2_v)

    w2r = [w2_v[j] for j in range(4)]
    zero16 = jnp.zeros((L,), jnp.float32)
    lane8 = lax.iota(jnp.int32, L) * 8

    def sub_chunk(k, carry):
        eb = base + k * SUB
        pltpu.sync_copy(row_hbm.at[pl.ds(eb, SUB)], rowv)
        pltpu.sync_copy(col_hbm.at[pl.ds(eb, SUB)], colv)
        pltpu.sync_copy(c_hbm.at[pl.ds(eb * 8, SUB * 8)], cv)

        def inner(i, carry2):
            r16 = rowv[pl.ds(i * L, L)]
            q16 = colv[pl.ds(i * L, L)]
            cb = lane8 + i * (8 * L) + jh * 4
            s_acc = zero16
            for jl in range(4):
                ag = plsc.load_gather(pa[jl], [r16])
                bg = plsc.load_gather(pb[jl], [q16])
                cg = plsc.load_gather(cv, [cb + jl])
                pre = jnp.maximum(ag + bg + cg, 0.0)
                s_acc = s_acc + pre * w2r[jl]
            sv[pl.ds(i * L, L)] = s_acc
            return carry2

        lax.fori_loop(0, SUB // L, inner, 0)

        pltpu.sync_copy(sv, out_hbm.at[pl.ds(jh * N_EDGES_C + eb, SUB)])

        return carry

    lax.fori_loop(0, NSUB, sub_chunk, 0)


_sc_call = functools.partial(
    pl.kernel,
    out_type=jax.ShapeDtypeStruct((2 * N_EDGES_C,), jnp.float32),
    mesh=plsc.VectorSubcoreMesh(core_axis_name="c", subcore_axis_name="s"),
    compiler_params=pltpu.CompilerParams(needs_layout_passes=False),
    scratch_types=(
        [pltpu.VMEM((N_NODES_C,), jnp.float32)] * 8
        + [
            pltpu.VMEM((SUB,), jnp.int32),
            pltpu.VMEM((SUB,), jnp.int32),
            pltpu.VMEM((SUB * 8,), jnp.float32),
            pltpu.VMEM((SUB,), jnp.float32),
            pltpu.VMEM((4, L), jnp.float32),
        ]
    ),
)


def _combine_body(s0_ref, s1_ref, b2_ref, o_ref):
    v = s0_ref[...] + s1_ref[...] + b2_ref[...]
    o_ref[...] = jax.nn.sigmoid(v)


def kernel(xn, edge_index, edge_attr, W1, b1, W2, b2):
    W1a = W1[:D_FEAT_C, :]
    W1b = W1[D_FEAT_C:2 * D_FEAT_C, :]
    W1c = W1[2 * D_FEAT_C:, :]
    Wab = jnp.concatenate([W1a, W1b], axis=1)

    ABt = pl.pallas_call(
        _mm_abt_body,
        out_shape=jax.ShapeDtypeStruct((2 * HID, N_NODES_C), jnp.float32),
    )(xn, Wab)

    ea_pack = edge_attr.reshape(N_EDGES_C // 8, 8 * D_EDGE_C)
    eye8 = jnp.eye(8, dtype=jnp.float32)
    w_bd = jnp.einsum("pq,kh->pkqh", eye8, W1c).reshape(8 * D_EDGE_C, 8 * HID)
    b_tile = jnp.tile(b1, 8).reshape(1, 8 * HID)
    C = pl.pallas_call(
        _mm_c_body,
        grid=(N_EDGES_C // 8 // _C_BLK,),
        in_specs=[
            pl.BlockSpec((_C_BLK, 8 * D_EDGE_C), lambda i: (i, 0)),
            pl.BlockSpec((8 * D_EDGE_C, 8 * HID), lambda i: (0, 0)),
            pl.BlockSpec((1, 8 * HID), lambda i: (0, 0)),
        ],
        out_specs=pl.BlockSpec((_C_BLK, 8 * HID), lambda i: (i, 0)),
        out_shape=jax.ShapeDtypeStruct((N_EDGES_C // 8, 8 * HID), jnp.float32),
    )(ea_pack, w_bd, b_tile)

    row = edge_index[0].astype(jnp.int32)
    col = edge_index[1].astype(jnp.int32)
    c_flat = C.reshape(-1)
    w2b = jnp.broadcast_to(W2.reshape(HID, 1), (HID, L))

    s01 = _sc_call(_sc_edge_mlp)(ABt, row, col, c_flat, w2b)
    s01_2d = s01.reshape(80, 8000)

    out = pl.pallas_call(
        _combine_body,
        grid=(1,),
        in_specs=[
            pl.BlockSpec((40, 8000), lambda i: (0, 0)),
            pl.BlockSpec((40, 8000), lambda i: (1, 0)),
            pl.BlockSpec((1, 1), lambda i: (0, 0)),
        ],
        out_specs=pl.BlockSpec((40, 8000), lambda i: (0, 0)),
        out_shape=jax.ShapeDtypeStruct((40, 8000), jnp.float32),
    )(s01_2d, s01_2d, b2.reshape(1, 1))
    return out.reshape(N_EDGES_C, 1)
